# initial kernel scaffold (unmeasured)
import jax
import jax.numpy as jnp
from jax import lax
from jax.experimental import pallas as pl
from jax.experimental.pallas import tpu as pltpu

T = 1024
D = 2048
VL = 16384
CH = 2048
NCH = VL // CH


def kernel(x, W):
    def body(x_ref, w_hbm, out_hbm, e_recv_hbm, e_loc_hbm,
             x_bf, w_vmem, e_send, proc, out_stage,
             send_sems, recv_sems, w_sem, loc_sem, out_sem):
        my_x = lax.axis_index("x")
        my_y = lax.axis_index("y")
        peer = (my_x, 1 - my_y)

        barrier = pltpu.get_barrier_semaphore()
        pl.semaphore_signal(
            barrier, inc=1, device_id=peer,
            device_id_type=pl.DeviceIdType.MESH,
        )
        pl.semaphore_wait(barrier, 1)

        x_bf[...] = x_ref[...].astype(jnp.bfloat16)

        s_l = jnp.zeros((T, 1), jnp.float32)
        descs = []
        for j in range(NCH):
            w_cp = pltpu.make_async_copy(
                w_hbm.at[:, pl.ds(j * CH, CH)], w_vmem, w_sem)
            w_cp.start()
            w_cp.wait()
            logits = jnp.dot(
                x_bf[...], w_vmem[...].astype(jnp.bfloat16),
                preferred_element_type=jnp.float32)
            e = jnp.exp(logits)
            s_l = s_l + jnp.sum(e, axis=1, keepdims=True)
            slot = j % 2
            if j >= 2:
                descs[j - 2].wait_send()
            e_send[slot, :, :] = e.astype(jnp.bfloat16)
            lc = pltpu.make_async_copy(
                e_send.at[slot], e_loc_hbm.at[j], loc_sem)
            lc.start()
            rdma = pltpu.make_async_remote_copy(
                src_ref=e_send.at[slot],
                dst_ref=e_recv_hbm.at[j],
                send_sem=send_sems.at[j],
                recv_sem=recv_sems.at[j],
                device_id=peer,
                device_id_type=pl.DeviceIdType.MESH,
            )
            rdma.start()
            descs.append(rdma)
            lc.wait()
        descs[NCH - 2].wait_send()
        descs[NCH - 1].wait_send()

        s_r = jnp.zeros((T, 1), jnp.float32)
        for j in range(NCH):
            descs[j].wait_recv()
            rc = pltpu.make_async_copy(e_recv_hbm.at[j], proc, loc_sem)
            rc.start()
            rc.wait()
            s_r = s_r + jnp.sum(
                proc[...].astype(jnp.float32), axis=1, keepdims=True)

        inv = 1.0 / (s_l + s_r)

        for half in range(2):
            src = e_loc_hbm if half == 0 else e_recv_hbm
            base = (my_y if half == 0 else 1 - my_y) * VL
            for j in range(NCH):
                rc = pltpu.make_async_copy(src.at[j], proc, loc_sem)
                rc.start()
                rc.wait()
                out_stage[...] = proc[...].astype(jnp.float32) * inv
                oc = pltpu.make_async_copy(
                    out_stage, out_hbm.at[:, pl.ds(base + j * CH, CH)],
                    out_sem)
                oc.start()
                oc.wait()

    out_shape = (
        jax.ShapeDtypeStruct((T, 2 * VL), jnp.float32),
        jax.ShapeDtypeStruct((NCH, T, CH), jnp.bfloat16),
        jax.ShapeDtypeStruct((NCH, T, CH), jnp.bfloat16),
    )
    out, _, _ = pl.pallas_call(
        body,
        out_shape=out_shape,
        in_specs=[
            pl.BlockSpec(memory_space=pltpu.VMEM),
            pl.BlockSpec(memory_space=pltpu.ANY),
        ],
        out_specs=[
            pl.BlockSpec(memory_space=pltpu.ANY),
            pl.BlockSpec(memory_space=pltpu.ANY),
            pl.BlockSpec(memory_space=pltpu.ANY),
        ],
        scratch_shapes=[
            pltpu.VMEM((T, D), jnp.bfloat16),
            pltpu.VMEM((D, CH), jnp.float32),
            pltpu.VMEM((2, T, CH), jnp.bfloat16),
            pltpu.VMEM((T, CH), jnp.bfloat16),
            pltpu.VMEM((T, CH), jnp.float32),
            pltpu.SemaphoreType.DMA((NCH,)),
            pltpu.SemaphoreType.DMA((NCH,)),
            pltpu.SemaphoreType.DMA,
            pltpu.SemaphoreType.DMA,
            pltpu.SemaphoreType.DMA,
        ],
        compiler_params=pltpu.CompilerParams(collective_id=0),
    )(x, W)
    return out


# baseline (device time: 586558 ns/iter reference)
import jax
import jax.numpy as jnp
from jax import lax
from jax.experimental import pallas as pl
from jax.experimental.pallas import tpu as pltpu

T = 1024
D = 2048
VL = 16384
CH = 2048
NCH = VL // CH


def kernel(x, W):
    def body(x_ref, w_hbm, out_hbm, e_recv_hbm, e_loc_hbm,
             x_bf, w_vmem, e_send, proc, out_stage,
             send_sems, recv_sems, w_sem, loc_sem, out_sem):
        my_x = lax.axis_index("x")
        my_y = lax.axis_index("y")
        peer = (my_x, 1 - my_y)

        barrier = pltpu.get_barrier_semaphore()
        pl.semaphore_signal(
            barrier, inc=1, device_id=peer,
            device_id_type=pl.DeviceIdType.MESH,
        )
        pl.semaphore_wait(barrier, 1)

        x_bf[...] = x_ref[...].astype(jnp.bfloat16)

        s_l = jnp.zeros((T, 1), jnp.float32)
        descs = []
        for j in range(NCH):
            w_cp = pltpu.make_async_copy(
                w_hbm.at[:, pl.ds(j * CH, CH)], w_vmem, w_sem)
            w_cp.start()
            w_cp.wait()
            logits = jnp.dot(
                x_bf[...], w_vmem[...].astype(jnp.bfloat16),
                preferred_element_type=jnp.float32)
            e = jnp.exp(logits)
            s_l = s_l + jnp.sum(e, axis=1, keepdims=True)
            slot = j % 2
            if j >= 2:
                descs[j - 2].wait_send()
            e_send[slot, :, :] = e.astype(jnp.bfloat16)
            lc = pltpu.make_async_copy(
                e_send.at[slot], e_loc_hbm.at[j], loc_sem)
            lc.start()
            rdma = pltpu.make_async_remote_copy(
                src_ref=e_send.at[slot],
                dst_ref=e_recv_hbm.at[j],
                send_sem=send_sems.at[j],
                recv_sem=recv_sems.at[j],
                device_id=peer,
                device_id_type=pl.DeviceIdType.MESH,
            )
            rdma.start()
            descs.append(rdma)
            lc.wait()
        descs[NCH - 2].wait_send()
        descs[NCH - 1].wait_send()

        s_r = jnp.zeros((T, 1), jnp.float32)
        for j in range(NCH):
            descs[j].wait_recv()
            rc = pltpu.make_async_copy(e_recv_hbm.at[j], proc, loc_sem)
            rc.start()
            rc.wait()
            s_r = s_r + jnp.sum(
                proc[...].astype(jnp.float32), axis=1, keepdims=True)

        inv = 1.0 / (s_l + s_r)

        for half in range(2):
            src = e_loc_hbm if half == 0 else e_recv_hbm
            base = (my_y if half == 0 else 1 - my_y) * VL
            for j in range(NCH):
                rc = pltpu.make_async_copy(src.at[j], proc, loc_sem)
                rc.start()
                rc.wait()
                out_stage[...] = proc[...].astype(jnp.float32) * inv
                oc = pltpu.make_async_copy(
                    out_stage, out_hbm.at[:, pl.ds(base + j * CH, CH)],
                    out_sem)
                oc.start()
                oc.wait()

    out_shape = (
        jax.ShapeDtypeStruct((T, 2 * VL), jnp.float32),
        jax.ShapeDtypeStruct((NCH, T, CH), jnp.bfloat16),
        jax.ShapeDtypeStruct((NCH, T, CH), jnp.bfloat16),
    )
    out, _, _ = pl.pallas_call(
        body,
        out_shape=out_shape,
        in_specs=[
            pl.BlockSpec(memory_space=pltpu.VMEM),
            pl.BlockSpec(memory_space=pl.ANY),
        ],
        out_specs=[
            pl.BlockSpec(memory_space=pl.ANY),
            pl.BlockSpec(memory_space=pl.ANY),
            pl.BlockSpec(memory_space=pl.ANY),
        ],
        scratch_shapes=[
            pltpu.VMEM((T, D), jnp.bfloat16),
            pltpu.VMEM((D, CH), jnp.float32),
            pltpu.VMEM((2, T, CH), jnp.bfloat16),
            pltpu.VMEM((T, CH), jnp.bfloat16),
            pltpu.VMEM((T, CH), jnp.float32),
            pltpu.SemaphoreType.DMA((NCH,)),
            pltpu.SemaphoreType.DMA((NCH,)),
            pltpu.SemaphoreType.DMA,
            pltpu.SemaphoreType.DMA,
            pltpu.SemaphoreType.DMA,
        ],
        compiler_params=pltpu.CompilerParams(
            collective_id=0,
            vmem_limit_bytes=100 * 1024 * 1024,
        ),
    )(x, W)
    return out


# device time: 472295 ns/iter; 1.2419x vs baseline; 1.2419x over previous
import jax
import jax.numpy as jnp
from jax import lax
from jax.experimental import pallas as pl
from jax.experimental.pallas import tpu as pltpu

T = 1024
D = 2048
VL = 16384
CH = 1024
NCH = VL // CH
K_EAGER = 4


def kernel(x, W):
    def body(x_ref, w_hbm, out_hbm, e_recv_hbm, e_loc_hbm,
             x_bf, w_vmem, proc, out_stage, s_send, s_recv,
             send_sems, recv_sems, s_send_sem, s_recv_sem,
             w_sems, st_sems, rd_sems, out_sems):
        my_x = lax.axis_index("x")
        my_y = lax.axis_index("y")
        peer = (my_x, 1 - my_y)

        barrier = pltpu.get_barrier_semaphore()
        pl.semaphore_signal(
            barrier, inc=1, device_id=peer,
            device_id_type=pl.DeviceIdType.MESH,
        )
        pl.semaphore_wait(barrier, 1)

        x_bf[...] = x_ref[...].astype(jnp.bfloat16)

        def w_load(j, slot):
            return pltpu.make_async_copy(
                w_hbm.at[:, pl.ds(j * CH, CH)], w_vmem.at[slot],
                w_sems.at[slot])

        w_load(0, 0).start()
        s_l = jnp.zeros((T, 1), jnp.float32)
        descs = []
        stash = []
        stash_waited = []
        for j in range(NCH):
            slot = j % 2
            if j + 1 < NCH:
                w_load(j + 1, 1 - slot).start()
            w_load(j, slot).wait()
            logits = jnp.dot(
                x_bf[...], w_vmem[slot].astype(jnp.bfloat16),
                preferred_element_type=jnp.float32)
            e = jnp.exp(logits)
            s_l = s_l + jnp.sum(e, axis=1, keepdims=True)
            if j >= 2 and not stash_waited[j - 2]:
                stash[j - 2].wait()
                stash_waited[j - 2] = True
            proc[slot, :, :] = e.astype(jnp.bfloat16)
            st = pltpu.make_async_copy(
                proc.at[slot], e_loc_hbm.at[j], st_sems.at[slot])
            st.start()
            stash.append(st)
            stash_waited.append(False)
            rdma = pltpu.make_async_remote_copy(
                src_ref=e_loc_hbm.at[j],
                dst_ref=e_recv_hbm.at[j],
                send_sem=send_sems.at[j],
                recv_sem=recv_sems.at[j],
                device_id=peer,
                device_id_type=pl.DeviceIdType.MESH,
            )
            descs.append(rdma)
            if j < K_EAGER:
                st.wait()
                stash_waited[j] = True
                rdma.start()
        for j in range(NCH):
            if not stash_waited[j]:
                stash[j].wait()
                stash_waited[j] = True

        s_send[...] = jnp.broadcast_to(s_l, (T, 8))
        s_rdma = pltpu.make_async_remote_copy(
            src_ref=s_send, dst_ref=s_recv,
            send_sem=s_send_sem, recv_sem=s_recv_sem,
            device_id=peer, device_id_type=pl.DeviceIdType.MESH,
        )
        s_rdma.start()
        for j in range(K_EAGER, NCH):
            descs[j].start()

        s_rdma.wait_recv()
        inv = 1.0 / (s_l + s_recv[:, 0:1])

        ocs = []
        for half in range(2):
            base = (my_y if half == 0 else 1 - my_y) * VL
            src = e_loc_hbm if half == 0 else e_recv_hbm
            for j in range(NCH):
                slot = j % 2
                if half == 1:
                    descs[j].wait_recv()
                rc = pltpu.make_async_copy(
                    src.at[j], proc.at[slot], rd_sems.at[slot])
                rc.start()
                rc.wait()
                if len(ocs) >= 2:
                    ocs[len(ocs) - 2].wait()
                out_stage[slot, :, :] = (
                    proc[slot].astype(jnp.float32) * inv)
                oc = pltpu.make_async_copy(
                    out_stage.at[slot],
                    out_hbm.at[:, pl.ds(base + j * CH, CH)],
                    out_sems.at[slot])
                oc.start()
                ocs.append(oc)
        ocs[-2].wait()
        ocs[-1].wait()

        for j in range(NCH):
            descs[j].wait_send()
        s_rdma.wait_send()

    out_shape = (
        jax.ShapeDtypeStruct((T, 2 * VL), jnp.float32),
        jax.ShapeDtypeStruct((NCH, T, CH), jnp.bfloat16),
        jax.ShapeDtypeStruct((NCH, T, CH), jnp.bfloat16),
    )
    out, _, _ = pl.pallas_call(
        body,
        out_shape=out_shape,
        in_specs=[
            pl.BlockSpec(memory_space=pltpu.VMEM),
            pl.BlockSpec(memory_space=pl.ANY),
        ],
        out_specs=[
            pl.BlockSpec(memory_space=pl.ANY),
            pl.BlockSpec(memory_space=pl.ANY),
            pl.BlockSpec(memory_space=pl.ANY),
        ],
        scratch_shapes=[
            pltpu.VMEM((T, D), jnp.bfloat16),
            pltpu.VMEM((2, D, CH), jnp.float32),
            pltpu.VMEM((2, T, CH), jnp.bfloat16),
            pltpu.VMEM((2, T, CH), jnp.float32),
            pltpu.VMEM((T, 8), jnp.float32),
            pltpu.VMEM((T, 8), jnp.float32),
            pltpu.SemaphoreType.DMA((NCH,)),
            pltpu.SemaphoreType.DMA((NCH,)),
            pltpu.SemaphoreType.DMA,
            pltpu.SemaphoreType.DMA,
            pltpu.SemaphoreType.DMA((2,)),
            pltpu.SemaphoreType.DMA((2,)),
            pltpu.SemaphoreType.DMA((2,)),
            pltpu.SemaphoreType.DMA((2,)),
        ],
        compiler_params=pltpu.CompilerParams(
            collective_id=0,
            vmem_limit_bytes=100 * 1024 * 1024,
        ),
    )(x, W)
    return out
